# Initial kernel scaffold; baseline (speedup 1.0000x reference)
import jax
import jax.numpy as jnp
from jax.experimental import pallas as pl


def kernel(x, edge_index, batch, Wl1, Wr1, b1, Wl2, Wr2, b2, Wl3, Wr3, b3, Wl4, Wr4, b4):
    raise NotImplementedError("write your pallas kernel here")


# speedup vs baseline: 9.8645x; 9.8645x over previous
import functools

import jax
import jax.numpy as jnp
from jax import lax
from jax.experimental import pallas as pl
from jax.experimental.pallas import tpu as pltpu
from jax.experimental.pallas import tpu_sc as plsc

_F32 = jnp.float32
_NC, _NS = 2, 16
_NW = _NC * _NS
_CH = 128
_W = 16
_G = 64


@functools.lru_cache(maxsize=None)
def _make_seg_sum(n_nodes, k_chunks, gather):
    zr = next(d for d in range(min(1024, n_nodes), 7, -1)
              if n_nodes % d == 0 and d % 8 == 0)
    nchunks = n_nodes // zr
    per_tile = -(-nchunks // _NS)
    n_acc = n_nodes + 8
    mesh = plsc.VectorSubcoreMesh(core_axis_name="c", subcore_axis_name="s")

    @functools.partial(
        pl.kernel,
        out_type=jax.ShapeDtypeStruct((_NC, n_nodes, _W), _F32),
        mesh=mesh,
        scratch_types=[
            pltpu.VMEM((k_chunks, _CH), jnp.int32),
            pltpu.VMEM((k_chunks, _CH), jnp.int32),
            pltpu.VMEM((_CH, _W), _F32),
            pltpu.VMEM((zr, _W), _F32),
            pltpu.VMEM_SHARED((n_acc, _W), _F32),
            pltpu.SemaphoreType.DMA,
        ],
        compiler_params=pltpu.CompilerParams(use_tc_tiling_on_sc=False),
    )
    def seg_sum(table, src, dst, out, idx_s, idx_d, rows, zbuf, acc, sem):
        cid = lax.axis_index("c")
        sid = lax.axis_index("s")
        wid = cid * _NS + sid

        zv = jnp.zeros((16,), _F32)

        def _z(i, _):
            zbuf[i, pl.ds(0, 16)] = zv
            return 0

        lax.fori_loop(0, zr, _z, 0)

        if not gather:
            ov = jnp.ones((16,), _F32)

            def _o(i, _):
                rows[i, pl.ds(0, 16)] = ov
                return 0

            lax.fori_loop(0, _CH, _o, 0)

        def _zero(i, _):
            ch = i * _NS + sid

            @pl.when(ch < nchunks)
            def _():
                pltpu.sync_copy(zbuf, acc.at[pl.ds(ch * zr, zr)])
            return 0

        lax.fori_loop(0, per_tile, _zero, 0)
        plsc.subcore_barrier()

        if gather:
            pltpu.sync_copy(src.at[wid], idx_s)
        pltpu.sync_copy(dst.at[wid], idx_d)

        def _edge(j, _):
            if gather:
                pltpu.async_copy(table.at[idx_s.at[j]], rows, sem).wait()
            pltpu.sync_copy(rows, acc.at[idx_d.at[j]], add=True)
            return 0

        lax.fori_loop(0, k_chunks, _edge, 0)
        plsc.subcore_barrier()

        def _drain(i, _):
            ch = i * _NS + sid

            @pl.when(ch < nchunks)
            def _():
                r0 = ch * zr
                pltpu.sync_copy(acc.at[pl.ds(r0, zr)], out.at[cid, pl.ds(r0, zr)])
            return 0

        lax.fori_loop(0, per_tile, _drain, 0)

    return seg_sum


def _full(shape):
    return pl.BlockSpec(shape, lambda i: tuple(0 for _ in shape))


def _row(blk, w):
    return pl.BlockSpec((blk, w), lambda i: (i, 0))


def _agg(blk):
    return pl.BlockSpec((2, blk, _W), lambda i: (0, i, 0))


@functools.lru_cache(maxsize=None)
def _make_tc1(n, blk):
    def body(x_ref, wl_ref, wr_ref, b_ref, y_ref, z_ref):
        xb = x_ref[...]
        y_ref[...] = jnp.dot(xb, wl_ref[...], preferred_element_type=_F32)
        z_ref[...] = jnp.dot(xb, wr_ref[...], preferred_element_type=_F32) + b_ref[...]

    return pl.pallas_call(
        body,
        grid=(n // blk,),
        in_specs=[_row(blk, 56), _full((56, 16)), _full((56, 16)), _full((1, 16))],
        out_specs=[_row(blk, 16), _row(blk, 16)],
        out_shape=[jax.ShapeDtypeStruct((n, 16), _F32),
                   jax.ShapeDtypeStruct((n, 16), _F32)],
    )


@functools.lru_cache(maxsize=None)
def _make_tc2(n, blk):
    def body(a_ref, c_ref, z_ref, h_ref, inv_ref):
        a = a_ref[0] + a_ref[1]
        cnt = c_ref[0, :, 0:1] + c_ref[1, :, 0:1]
        inv = 1.0 / jnp.maximum(cnt, 1.0)
        h_ref[...] = jnp.maximum(a * inv + z_ref[...], 0.0)
        inv_ref[...] = inv

    return pl.pallas_call(
        body,
        grid=(n // blk,),
        in_specs=[_agg(blk), _agg(blk), _row(blk, 16)],
        out_specs=[_row(blk, 16), _row(blk, 1)],
        out_shape=[jax.ShapeDtypeStruct((n, 16), _F32),
                   jax.ShapeDtypeStruct((n, 1), _F32)],
    )


@functools.lru_cache(maxsize=None)
def _make_tc3(n, blk):
    def body(a_ref, inv_ref, h_ref, wl_ref, wr_ref, b_ref, oa_ref, ob_ref):
        mean = (a_ref[0] + a_ref[1]) * inv_ref[...]
        o = jnp.maximum(
            jnp.dot(mean, wl_ref[...], preferred_element_type=_F32)
            + jnp.dot(h_ref[...], wr_ref[...], preferred_element_type=_F32)
            + b_ref[...], 0.0)
        oa_ref[...] = o[:, :16]
        ob_ref[...] = o[:, 16:]

    return pl.pallas_call(
        body,
        grid=(n // blk,),
        in_specs=[_agg(blk), _row(blk, 1), _row(blk, 16),
                  _full((16, 32)), _full((16, 32)), _full((1, 32))],
        out_specs=[_row(blk, 16), _row(blk, 16)],
        out_shape=[jax.ShapeDtypeStruct((n, 16), _F32),
                   jax.ShapeDtypeStruct((n, 16), _F32)],
    )


@functools.lru_cache(maxsize=None)
def _make_tc4(n, blk):
    def body(aa_ref, ab_ref, inv_ref, ha_ref, hb_ref, wl3_ref, wr3_ref,
             b3_ref, wl4_ref, wr4_ref, b4_ref, y_ref, z_ref):
        inv = inv_ref[...]
        mean = jnp.concatenate(
            [(aa_ref[0] + aa_ref[1]) * inv, (ab_ref[0] + ab_ref[1]) * inv],
            axis=1)
        h2 = jnp.concatenate([ha_ref[...], hb_ref[...]], axis=1)
        h3 = jnp.maximum(
            jnp.dot(mean, wl3_ref[...], preferred_element_type=_F32)
            + jnp.dot(h2, wr3_ref[...], preferred_element_type=_F32)
            + b3_ref[...], 0.0)
        pad = jnp.zeros((blk, 14), _F32)
        y_ref[...] = jnp.concatenate(
            [jnp.dot(h3, wl4_ref[...], preferred_element_type=_F32), pad], axis=1)
        z_ref[...] = jnp.concatenate(
            [jnp.dot(h3, wr4_ref[...], preferred_element_type=_F32) + b4_ref[...],
             pad], axis=1)

    return pl.pallas_call(
        body,
        grid=(n // blk,),
        in_specs=[_agg(blk), _agg(blk), _row(blk, 1), _row(blk, 16),
                  _row(blk, 16), _full((32, 64)), _full((32, 64)),
                  _full((1, 64)), _full((64, 2)), _full((64, 2)), _full((1, 2))],
        out_specs=[_row(blk, 16), _row(blk, 16)],
        out_shape=[jax.ShapeDtypeStruct((n, 16), _F32),
                   jax.ShapeDtypeStruct((n, 16), _F32)],
    )


@functools.lru_cache(maxsize=None)
def _make_tc5(n, blk):
    nb = n // blk

    def body(a_ref, inv_ref, z4_ref, bt_ref, o_ref, acc_ref):
        i = pl.program_id(0)
        a = a_ref[0] + a_ref[1]
        h4 = jnp.maximum(a[:, :2] * inv_ref[...] + z4_ref[:, :2], 0.0)
        hc = jnp.concatenate(
            [h4, jnp.ones((blk, 1), _F32), jnp.zeros((blk, 5), _F32)], axis=1)
        bt = bt_ref[...][0, 0]
        oh = (bt[:, None] == lax.broadcasted_iota(jnp.int32, (blk, _G), 1)
              ).astype(_F32)
        part = lax.dot_general(oh, hc, (((0,), (0,)), ((), ())),
                               preferred_element_type=_F32)

        @pl.when(i == 0)
        def _():
            acc_ref[...] = part

        @pl.when(i > 0)
        def _():
            acc_ref[...] = acc_ref[...] + part

        @pl.when(i == nb - 1)
        def _():
            acc = acc_ref[...]
            pooled = acc[:, :2] / jnp.maximum(acc[:, 2:3], 1.0)
            m = jnp.max(pooled, axis=1, keepdims=True)
            e = jnp.exp(pooled - m)
            o_ref[...] = e / jnp.sum(e, axis=1, keepdims=True)

    return pl.pallas_call(
        body,
        grid=(nb,),
        in_specs=[_agg(blk), _row(blk, 1), _row(blk, 16),
                  pl.BlockSpec((1, 1, blk), lambda i: (i, 0, 0))],
        out_specs=pl.BlockSpec((_G, 2), lambda i: (0, 0)),
        out_shape=jax.ShapeDtypeStruct((_G, 2), _F32),
        scratch_shapes=[pltpu.VMEM((_G, 8), _F32)],
    )


def kernel(x, edge_index, batch, Wl1, Wr1, b1, Wl2, Wr2, b2,
           Wl3, Wr3, b3, Wl4, Wr4, b4):
    n = x.shape[0]
    e = edge_index.shape[1]
    blk = next(d for d in range(min(2048, n), 7, -1) if n % d == 0 and d % 8 == 0)
    nb = n // blk

    k = -(-e // (_NW * _CH))
    ep = _NW * k * _CH
    src = jnp.concatenate(
        [edge_index[0], jnp.zeros((ep - e,), jnp.int32)]).reshape(_NW, k, _CH)
    dst = jnp.concatenate(
        [edge_index[1], jnp.full((ep - e,), n, jnp.int32)]).reshape(_NW, k, _CH)

    seg = _make_seg_sum(n, k, True)
    cnt = _make_seg_sum(n, k, False)

    y1, z1 = _make_tc1(n, blk)(x, Wl1, Wr1, b1.reshape(1, -1))
    c = cnt(y1, src, dst)
    a1 = seg(y1, src, dst)
    h1, inv = _make_tc2(n, blk)(a1, c, z1)
    a2 = seg(h1, src, dst)
    h2a, h2b = _make_tc3(n, blk)(a2, inv, h1, Wl2, Wr2, b2.reshape(1, -1))
    a3a = seg(h2a, src, dst)
    a3b = seg(h2b, src, dst)
    y4, z4 = _make_tc4(n, blk)(a3a, a3b, inv, h2a, h2b, Wl3, Wr3,
                               b3.reshape(1, -1), Wl4, Wr4, b4.reshape(1, -1))
    a4 = seg(y4, src, dst)
    return _make_tc5(n, blk)(a4, inv, z4, batch.reshape(nb, 1, blk))


# speedup vs baseline: 13.4893x; 1.3675x over previous
import functools

import jax
import jax.numpy as jnp
from jax import lax
from jax.experimental import pallas as pl
from jax.experimental.pallas import tpu as pltpu
from jax.experimental.pallas import tpu_sc as plsc

_F32 = jnp.float32
_NC, _NS = 2, 16
_NW = _NC * _NS
_CH = 128
_W = 16
_G = 64


@functools.lru_cache(maxsize=None)
def _make_seg_sum(n_nodes, k_chunks, gather):
    zr = next(d for d in range(min(1024, n_nodes), 7, -1)
              if n_nodes % d == 0 and d % 8 == 0)
    nchunks = n_nodes // zr
    per_tile = -(-nchunks // _NS)
    n_acc = n_nodes + 8
    mesh = plsc.VectorSubcoreMesh(core_axis_name="c", subcore_axis_name="s")

    @functools.partial(
        pl.kernel,
        out_type=jax.ShapeDtypeStruct((_NC, n_nodes, _W), _F32),
        mesh=mesh,
        scratch_types=[
            pltpu.VMEM((k_chunks, _CH), jnp.int32),
            pltpu.VMEM((k_chunks, _CH), jnp.int32),
            pltpu.VMEM((_CH, _W), _F32),
            pltpu.VMEM((_CH, _W), _F32),
            pltpu.VMEM((zr, _W), _F32),
            pltpu.VMEM_SHARED((n_acc, _W), _F32),
            pltpu.SemaphoreType.DMA,
            pltpu.SemaphoreType.DMA,
        ],
        compiler_params=pltpu.CompilerParams(use_tc_tiling_on_sc=False),
    )
    def seg_sum(table, src, dst, out, idx_s, idx_d, rows0, rows1, zbuf, acc,
                sem0, sem1):
        cid = lax.axis_index("c")
        sid = lax.axis_index("s")
        wid = cid * _NS + sid

        zv = jnp.zeros((16,), _F32)

        def _z(i, _):
            zbuf[i, pl.ds(0, 16)] = zv
            return 0

        lax.fori_loop(0, zr, _z, 0)

        if not gather:
            ov = jnp.ones((16,), _F32)

            def _o(i, _):
                rows0[i, pl.ds(0, 16)] = ov
                return 0

            lax.fori_loop(0, _CH, _o, 0)

        def _zero(i, _):
            ch = i * _NS + sid

            @pl.when(ch < nchunks)
            def _():
                pltpu.sync_copy(zbuf, acc.at[pl.ds(ch * zr, zr)])
            return 0

        lax.fori_loop(0, per_tile, _zero, 0)
        plsc.subcore_barrier()

        if gather:
            pltpu.sync_copy(src.at[wid], idx_s)
        pltpu.sync_copy(dst.at[wid], idx_d)

        if gather:
            pltpu.async_copy(table.at[idx_s.at[0]], rows0, sem0)
            if k_chunks > 1:
                pltpu.async_copy(table.at[idx_s.at[1]], rows1, sem1)

            def _one(c, rows, sem):
                pltpu.make_async_copy(table.at[idx_s.at[c]], rows, sem).wait()
                pltpu.sync_copy(rows, acc.at[idx_d.at[c]], add=True)

                @pl.when(c + 2 < k_chunks)
                def _():
                    pltpu.async_copy(table.at[idx_s.at[c + 2]], rows, sem)

            def _edge(bi, _):
                _one(2 * bi, rows0, sem0)
                _one(2 * bi + 1, rows1, sem1)
                return 0

            lax.fori_loop(0, k_chunks // 2, _edge, 0)
            if k_chunks % 2:
                c = k_chunks - 1
                _one(c, (rows0, rows1)[c % 2], (sem0, sem1)[c % 2])
        else:

            def _edge(j, _):
                pltpu.sync_copy(rows0, acc.at[idx_d.at[j]], add=True)
                return 0

            lax.fori_loop(0, k_chunks, _edge, 0)
        plsc.subcore_barrier()

        def _drain(i, _):
            ch = i * _NS + sid

            @pl.when(ch < nchunks)
            def _():
                r0 = ch * zr
                pltpu.sync_copy(acc.at[pl.ds(r0, zr)], out.at[cid, pl.ds(r0, zr)])
            return 0

        lax.fori_loop(0, per_tile, _drain, 0)

    return seg_sum


def _full(shape):
    return pl.BlockSpec(shape, lambda i: tuple(0 for _ in shape))


def _row(blk, w):
    return pl.BlockSpec((blk, w), lambda i: (i, 0))


def _agg(blk):
    return pl.BlockSpec((2, blk, _W), lambda i: (0, i, 0))


@functools.lru_cache(maxsize=None)
def _make_tc1(n, blk):
    def body(x_ref, wl_ref, wr_ref, b_ref, y_ref, z_ref):
        xb = x_ref[...]
        y_ref[...] = jnp.dot(xb, wl_ref[...], preferred_element_type=_F32)
        z_ref[...] = jnp.dot(xb, wr_ref[...], preferred_element_type=_F32) + b_ref[...]

    return pl.pallas_call(
        body,
        grid=(n // blk,),
        in_specs=[_row(blk, 56), _full((56, 16)), _full((56, 16)), _full((1, 16))],
        out_specs=[_row(blk, 16), _row(blk, 16)],
        out_shape=[jax.ShapeDtypeStruct((n, 16), _F32),
                   jax.ShapeDtypeStruct((n, 16), _F32)],
    )


@functools.lru_cache(maxsize=None)
def _make_tc2(n, blk):
    def body(a_ref, c_ref, z_ref, h_ref, inv_ref):
        a = a_ref[0] + a_ref[1]
        cnt = c_ref[0, :, 0:1] + c_ref[1, :, 0:1]
        inv = 1.0 / jnp.maximum(cnt, 1.0)
        h_ref[...] = jnp.maximum(a * inv + z_ref[...], 0.0)
        inv_ref[...] = inv

    return pl.pallas_call(
        body,
        grid=(n // blk,),
        in_specs=[_agg(blk), _agg(blk), _row(blk, 16)],
        out_specs=[_row(blk, 16), _row(blk, 1)],
        out_shape=[jax.ShapeDtypeStruct((n, 16), _F32),
                   jax.ShapeDtypeStruct((n, 1), _F32)],
    )


@functools.lru_cache(maxsize=None)
def _make_tc3(n, blk):
    def body(a_ref, inv_ref, h_ref, wl_ref, wr_ref, b_ref, oa_ref, ob_ref):
        mean = (a_ref[0] + a_ref[1]) * inv_ref[...]
        o = jnp.maximum(
            jnp.dot(mean, wl_ref[...], preferred_element_type=_F32)
            + jnp.dot(h_ref[...], wr_ref[...], preferred_element_type=_F32)
            + b_ref[...], 0.0)
        oa_ref[...] = o[:, :16]
        ob_ref[...] = o[:, 16:]

    return pl.pallas_call(
        body,
        grid=(n // blk,),
        in_specs=[_agg(blk), _row(blk, 1), _row(blk, 16),
                  _full((16, 32)), _full((16, 32)), _full((1, 32))],
        out_specs=[_row(blk, 16), _row(blk, 16)],
        out_shape=[jax.ShapeDtypeStruct((n, 16), _F32),
                   jax.ShapeDtypeStruct((n, 16), _F32)],
    )


@functools.lru_cache(maxsize=None)
def _make_tc4(n, blk):
    def body(aa_ref, ab_ref, inv_ref, ha_ref, hb_ref, wl3_ref, wr3_ref,
             b3_ref, wl4_ref, wr4_ref, b4_ref, y_ref, z_ref):
        inv = inv_ref[...]
        mean = jnp.concatenate(
            [(aa_ref[0] + aa_ref[1]) * inv, (ab_ref[0] + ab_ref[1]) * inv],
            axis=1)
        h2 = jnp.concatenate([ha_ref[...], hb_ref[...]], axis=1)
        h3 = jnp.maximum(
            jnp.dot(mean, wl3_ref[...], preferred_element_type=_F32)
            + jnp.dot(h2, wr3_ref[...], preferred_element_type=_F32)
            + b3_ref[...], 0.0)
        pad = jnp.zeros((blk, 14), _F32)
        y_ref[...] = jnp.concatenate(
            [jnp.dot(h3, wl4_ref[...], preferred_element_type=_F32), pad], axis=1)
        z_ref[...] = jnp.concatenate(
            [jnp.dot(h3, wr4_ref[...], preferred_element_type=_F32) + b4_ref[...],
             pad], axis=1)

    return pl.pallas_call(
        body,
        grid=(n // blk,),
        in_specs=[_agg(blk), _agg(blk), _row(blk, 1), _row(blk, 16),
                  _row(blk, 16), _full((32, 64)), _full((32, 64)),
                  _full((1, 64)), _full((64, 2)), _full((64, 2)), _full((1, 2))],
        out_specs=[_row(blk, 16), _row(blk, 16)],
        out_shape=[jax.ShapeDtypeStruct((n, 16), _F32),
                   jax.ShapeDtypeStruct((n, 16), _F32)],
    )


@functools.lru_cache(maxsize=None)
def _make_tc5(n, blk):
    nb = n // blk

    def body(a_ref, inv_ref, z4_ref, bt_ref, o_ref, acc_ref):
        i = pl.program_id(0)
        a = a_ref[0] + a_ref[1]
        h4 = jnp.maximum(a[:, :2] * inv_ref[...] + z4_ref[:, :2], 0.0)
        hc = jnp.concatenate(
            [h4, jnp.ones((blk, 1), _F32), jnp.zeros((blk, 5), _F32)], axis=1)
        bt = bt_ref[...][0, 0]
        oh = (bt[:, None] == lax.broadcasted_iota(jnp.int32, (blk, _G), 1)
              ).astype(_F32)
        part = lax.dot_general(oh, hc, (((0,), (0,)), ((), ())),
                               preferred_element_type=_F32)

        @pl.when(i == 0)
        def _():
            acc_ref[...] = part

        @pl.when(i > 0)
        def _():
            acc_ref[...] = acc_ref[...] + part

        @pl.when(i == nb - 1)
        def _():
            acc = acc_ref[...]
            pooled = acc[:, :2] / jnp.maximum(acc[:, 2:3], 1.0)
            m = jnp.max(pooled, axis=1, keepdims=True)
            e = jnp.exp(pooled - m)
            o_ref[...] = e / jnp.sum(e, axis=1, keepdims=True)

    return pl.pallas_call(
        body,
        grid=(nb,),
        in_specs=[_agg(blk), _row(blk, 1), _row(blk, 16),
                  pl.BlockSpec((1, 1, blk), lambda i: (i, 0, 0))],
        out_specs=pl.BlockSpec((_G, 2), lambda i: (0, 0)),
        out_shape=jax.ShapeDtypeStruct((_G, 2), _F32),
        scratch_shapes=[pltpu.VMEM((_G, 8), _F32)],
    )


def kernel(x, edge_index, batch, Wl1, Wr1, b1, Wl2, Wr2, b2,
           Wl3, Wr3, b3, Wl4, Wr4, b4):
    n = x.shape[0]
    e = edge_index.shape[1]
    blk = next(d for d in range(min(2048, n), 7, -1) if n % d == 0 and d % 8 == 0)
    nb = n // blk

    k = -(-e // (_NW * _CH))
    ep = _NW * k * _CH
    src = jnp.concatenate(
        [edge_index[0], jnp.zeros((ep - e,), jnp.int32)]).reshape(_NW, k, _CH)
    dst = jnp.concatenate(
        [edge_index[1], jnp.full((ep - e,), n, jnp.int32)]).reshape(_NW, k, _CH)

    seg = _make_seg_sum(n, k, True)
    cnt = _make_seg_sum(n, k, False)

    y1, z1 = _make_tc1(n, blk)(x, Wl1, Wr1, b1.reshape(1, -1))
    c = cnt(y1, src, dst)
    a1 = seg(y1, src, dst)
    h1, inv = _make_tc2(n, blk)(a1, c, z1)
    a2 = seg(h1, src, dst)
    h2a, h2b = _make_tc3(n, blk)(a2, inv, h1, Wl2, Wr2, b2.reshape(1, -1))
    a3a = seg(h2a, src, dst)
    a3b = seg(h2b, src, dst)
    y4, z4 = _make_tc4(n, blk)(a3a, a3b, inv, h2a, h2b, Wl3, Wr3,
                               b3.reshape(1, -1), Wl4, Wr4, b4.reshape(1, -1))
    a4 = seg(y4, src, dst)
    return _make_tc5(n, blk)(a4, inv, z4, batch.reshape(nb, 1, blk))


# speedup vs baseline: 15.8247x; 1.1731x over previous
import functools

import jax
import jax.numpy as jnp
from jax import lax
from jax.experimental import pallas as pl
from jax.experimental.pallas import tpu as pltpu
from jax.experimental.pallas import tpu_sc as plsc

_F32 = jnp.float32
_NC, _NS = 2, 16
_NW = _NC * _NS
_CH = 128
_W = 16
_DEPTH = 4
_G = 64


@functools.lru_cache(maxsize=None)
def _make_seg_sum(n_nodes, k_chunks, gather):
    zr = next(d for d in range(min(1024, n_nodes), 7, -1)
              if n_nodes % d == 0 and d % 8 == 0)
    nchunks = n_nodes // zr
    per_tile = -(-nchunks // _NS)
    n_acc = n_nodes + 8
    mesh = plsc.VectorSubcoreMesh(core_axis_name="c", subcore_axis_name="s")

    @functools.partial(
        pl.kernel,
        out_type=jax.ShapeDtypeStruct((_NC, n_nodes, _W), _F32),
        mesh=mesh,
        scratch_types=[
            pltpu.VMEM((k_chunks, _CH), jnp.int32),
            pltpu.VMEM((k_chunks, _CH), jnp.int32),
        ] + [pltpu.VMEM((_CH, _W), _F32)] * _DEPTH + [
            pltpu.VMEM((zr, _W), _F32),
            pltpu.VMEM_SHARED((n_acc, _W), _F32),
        ] + [pltpu.SemaphoreType.DMA] * _DEPTH,
        compiler_params=pltpu.CompilerParams(use_tc_tiling_on_sc=False),
    )
    def seg_sum(table, src, dst, out, idx_s, idx_d, *bufs):
        rows = bufs[:_DEPTH]
        zbuf = bufs[_DEPTH]
        acc = bufs[_DEPTH + 1]
        sems = bufs[_DEPTH + 2:]
        cid = lax.axis_index("c")
        sid = lax.axis_index("s")
        wid = cid * _NS + sid

        zv = jnp.zeros((16,), _F32)

        def _z(i, _):
            zbuf[i, pl.ds(0, 16)] = zv
            return 0

        lax.fori_loop(0, zr, _z, 0)

        if not gather:
            ov = jnp.ones((16,), _F32)

            def _o(i, _):
                rows[0][i, pl.ds(0, 16)] = ov
                return 0

            lax.fori_loop(0, _CH, _o, 0)

        def _zero(i, _):
            ch = i * _NS + sid

            @pl.when(ch < nchunks)
            def _():
                pltpu.sync_copy(zbuf, acc.at[pl.ds(ch * zr, zr)])
            return 0

        lax.fori_loop(0, per_tile, _zero, 0)
        plsc.subcore_barrier()

        if gather:
            pltpu.sync_copy(src.at[wid], idx_s)
        pltpu.sync_copy(dst.at[wid], idx_d)

        if gather:
            for p in range(min(_DEPTH, k_chunks)):
                pltpu.async_copy(table.at[idx_s.at[p]], rows[p], sems[p])

            def _one(c, p):
                pltpu.make_async_copy(
                    table.at[idx_s.at[c]], rows[p], sems[p]).wait()
                pltpu.sync_copy(rows[p], acc.at[idx_d.at[c]], add=True)

                @pl.when(c + _DEPTH < k_chunks)
                def _():
                    pltpu.async_copy(
                        table.at[idx_s.at[c + _DEPTH]], rows[p], sems[p])

            def _edge(bi, _):
                for p in range(_DEPTH):
                    _one(_DEPTH * bi + p, p)
                return 0

            lax.fori_loop(0, k_chunks // _DEPTH, _edge, 0)
            for c in range(k_chunks - k_chunks % _DEPTH, k_chunks):
                _one(c, c % _DEPTH)
        else:

            def _edge(j, _):
                pltpu.sync_copy(rows[0], acc.at[idx_d.at[j]], add=True)
                return 0

            lax.fori_loop(0, k_chunks, _edge, 0)
        plsc.subcore_barrier()

        def _drain(i, _):
            ch = i * _NS + sid

            @pl.when(ch < nchunks)
            def _():
                r0 = ch * zr
                pltpu.sync_copy(acc.at[pl.ds(r0, zr)], out.at[cid, pl.ds(r0, zr)])
            return 0

        lax.fori_loop(0, per_tile, _drain, 0)

    return seg_sum


def _full(shape):
    return pl.BlockSpec(shape, lambda i: tuple(0 for _ in shape))


def _row(blk, w):
    return pl.BlockSpec((blk, w), lambda i: (i, 0))


def _agg(blk):
    return pl.BlockSpec((2, blk, _W), lambda i: (0, i, 0))


@functools.lru_cache(maxsize=None)
def _make_tc1(n, blk):
    def body(x_ref, wl_ref, wr_ref, b_ref, y_ref, z_ref):
        xb = x_ref[...]
        y_ref[...] = jnp.dot(xb, wl_ref[...], preferred_element_type=_F32)
        z_ref[...] = jnp.dot(xb, wr_ref[...], preferred_element_type=_F32) + b_ref[...]

    return pl.pallas_call(
        body,
        grid=(n // blk,),
        in_specs=[_row(blk, 56), _full((56, 16)), _full((56, 16)), _full((1, 16))],
        out_specs=[_row(blk, 16), _row(blk, 16)],
        out_shape=[jax.ShapeDtypeStruct((n, 16), _F32),
                   jax.ShapeDtypeStruct((n, 16), _F32)],
    )


@functools.lru_cache(maxsize=None)
def _make_tc2(n, blk):
    def body(a_ref, c_ref, z_ref, h_ref, inv_ref):
        a = a_ref[0] + a_ref[1]
        cnt = c_ref[0, :, 0:1] + c_ref[1, :, 0:1]
        inv = 1.0 / jnp.maximum(cnt, 1.0)
        h_ref[...] = jnp.maximum(a * inv + z_ref[...], 0.0)
        inv_ref[...] = inv

    return pl.pallas_call(
        body,
        grid=(n // blk,),
        in_specs=[_agg(blk), _agg(blk), _row(blk, 16)],
        out_specs=[_row(blk, 16), _row(blk, 1)],
        out_shape=[jax.ShapeDtypeStruct((n, 16), _F32),
                   jax.ShapeDtypeStruct((n, 1), _F32)],
    )


@functools.lru_cache(maxsize=None)
def _make_tc3(n, blk):
    def body(a_ref, inv_ref, h_ref, wl_ref, wr_ref, b_ref, oa_ref, ob_ref):
        mean = (a_ref[0] + a_ref[1]) * inv_ref[...]
        o = jnp.maximum(
            jnp.dot(mean, wl_ref[...], preferred_element_type=_F32)
            + jnp.dot(h_ref[...], wr_ref[...], preferred_element_type=_F32)
            + b_ref[...], 0.0)
        oa_ref[...] = o[:, :16]
        ob_ref[...] = o[:, 16:]

    return pl.pallas_call(
        body,
        grid=(n // blk,),
        in_specs=[_agg(blk), _row(blk, 1), _row(blk, 16),
                  _full((16, 32)), _full((16, 32)), _full((1, 32))],
        out_specs=[_row(blk, 16), _row(blk, 16)],
        out_shape=[jax.ShapeDtypeStruct((n, 16), _F32),
                   jax.ShapeDtypeStruct((n, 16), _F32)],
    )


@functools.lru_cache(maxsize=None)
def _make_tc4(n, blk):
    def body(aa_ref, ab_ref, inv_ref, ha_ref, hb_ref, wl3_ref, wr3_ref,
             b3_ref, wl4_ref, wr4_ref, b4_ref, y_ref, z_ref):
        inv = inv_ref[...]
        mean = jnp.concatenate(
            [(aa_ref[0] + aa_ref[1]) * inv, (ab_ref[0] + ab_ref[1]) * inv],
            axis=1)
        h2 = jnp.concatenate([ha_ref[...], hb_ref[...]], axis=1)
        h3 = jnp.maximum(
            jnp.dot(mean, wl3_ref[...], preferred_element_type=_F32)
            + jnp.dot(h2, wr3_ref[...], preferred_element_type=_F32)
            + b3_ref[...], 0.0)
        pad = jnp.zeros((blk, 14), _F32)
        y_ref[...] = jnp.concatenate(
            [jnp.dot(h3, wl4_ref[...], preferred_element_type=_F32), pad], axis=1)
        z_ref[...] = jnp.concatenate(
            [jnp.dot(h3, wr4_ref[...], preferred_element_type=_F32) + b4_ref[...],
             pad], axis=1)

    return pl.pallas_call(
        body,
        grid=(n // blk,),
        in_specs=[_agg(blk), _agg(blk), _row(blk, 1), _row(blk, 16),
                  _row(blk, 16), _full((32, 64)), _full((32, 64)),
                  _full((1, 64)), _full((64, 2)), _full((64, 2)), _full((1, 2))],
        out_specs=[_row(blk, 16), _row(blk, 16)],
        out_shape=[jax.ShapeDtypeStruct((n, 16), _F32),
                   jax.ShapeDtypeStruct((n, 16), _F32)],
    )


@functools.lru_cache(maxsize=None)
def _make_tc5(n, blk):
    nb = n // blk

    def body(a_ref, inv_ref, z4_ref, bt_ref, o_ref, acc_ref):
        i = pl.program_id(0)
        a = a_ref[0] + a_ref[1]
        h4 = jnp.maximum(a[:, :2] * inv_ref[...] + z4_ref[:, :2], 0.0)
        hc = jnp.concatenate(
            [h4, jnp.ones((blk, 1), _F32), jnp.zeros((blk, 5), _F32)], axis=1)
        bt = bt_ref[...][0, 0]
        oh = (bt[:, None] == lax.broadcasted_iota(jnp.int32, (blk, _G), 1)
              ).astype(_F32)
        part = lax.dot_general(oh, hc, (((0,), (0,)), ((), ())),
                               preferred_element_type=_F32)

        @pl.when(i == 0)
        def _():
            acc_ref[...] = part

        @pl.when(i > 0)
        def _():
            acc_ref[...] = acc_ref[...] + part

        @pl.when(i == nb - 1)
        def _():
            acc = acc_ref[...]
            pooled = acc[:, :2] / jnp.maximum(acc[:, 2:3], 1.0)
            m = jnp.max(pooled, axis=1, keepdims=True)
            e = jnp.exp(pooled - m)
            o_ref[...] = e / jnp.sum(e, axis=1, keepdims=True)

    return pl.pallas_call(
        body,
        grid=(nb,),
        in_specs=[_agg(blk), _row(blk, 1), _row(blk, 16),
                  pl.BlockSpec((1, 1, blk), lambda i: (i, 0, 0))],
        out_specs=pl.BlockSpec((_G, 2), lambda i: (0, 0)),
        out_shape=jax.ShapeDtypeStruct((_G, 2), _F32),
        scratch_shapes=[pltpu.VMEM((_G, 8), _F32)],
    )


def kernel(x, edge_index, batch, Wl1, Wr1, b1, Wl2, Wr2, b2,
           Wl3, Wr3, b3, Wl4, Wr4, b4):
    n = x.shape[0]
    e = edge_index.shape[1]
    blk = next(d for d in range(min(2048, n), 7, -1) if n % d == 0 and d % 8 == 0)
    nb = n // blk

    k = -(-e // (_NW * _CH))
    ep = _NW * k * _CH
    src = jnp.concatenate(
        [edge_index[0], jnp.zeros((ep - e,), jnp.int32)]).reshape(_NW, k, _CH)
    dst = jnp.concatenate(
        [edge_index[1], jnp.full((ep - e,), n, jnp.int32)]).reshape(_NW, k, _CH)

    seg = _make_seg_sum(n, k, True)
    cnt = _make_seg_sum(n, k, False)

    y1, z1 = _make_tc1(n, blk)(x, Wl1, Wr1, b1.reshape(1, -1))
    c = cnt(y1, src, dst)
    a1 = seg(y1, src, dst)
    h1, inv = _make_tc2(n, blk)(a1, c, z1)
    a2 = seg(h1, src, dst)
    h2a, h2b = _make_tc3(n, blk)(a2, inv, h1, Wl2, Wr2, b2.reshape(1, -1))
    a3a = seg(h2a, src, dst)
    a3b = seg(h2b, src, dst)
    y4, z4 = _make_tc4(n, blk)(a3a, a3b, inv, h2a, h2b, Wl3, Wr3,
                               b3.reshape(1, -1), Wl4, Wr4, b4.reshape(1, -1))
    a4 = seg(y4, src, dst)
    return _make_tc5(n, blk)(a4, inv, z4, batch.reshape(nb, 1, blk))


# speedup vs baseline: 16.0462x; 1.0140x over previous
import functools

import jax
import jax.numpy as jnp
from jax import lax
from jax.experimental import pallas as pl
from jax.experimental.pallas import tpu as pltpu
from jax.experimental.pallas import tpu_sc as plsc

_F32 = jnp.float32
_NC, _NS = 2, 16
_NW = _NC * _NS
_CH = 128
_W = 16
_B = 8
_F = 4
_G = 64


@functools.lru_cache(maxsize=None)
def _make_seg_sum(n_nodes, k_chunks, gather):
    zr = next(d for d in range(min(256, n_nodes), 7, -1)
              if n_nodes % d == 0 and d % 8 == 0)
    nchunks = n_nodes // zr
    per_tile = -(-nchunks // _NS)
    n_acc = n_nodes + 8
    mesh = plsc.VectorSubcoreMesh(core_axis_name="c", subcore_axis_name="s")

    @functools.partial(
        pl.kernel,
        out_type=jax.ShapeDtypeStruct((_NC, n_nodes, _W), _F32),
        mesh=mesh,
        scratch_types=[
            pltpu.VMEM((k_chunks, _CH), jnp.int32),
            pltpu.VMEM((k_chunks, _CH), jnp.int32),
        ] + [pltpu.VMEM((_CH, _W), _F32)] * _B + [
            pltpu.VMEM((zr, _W), _F32),
            pltpu.VMEM_SHARED((n_acc, _W), _F32),
        ] + [pltpu.SemaphoreType.DMA] * (2 * _F),
        compiler_params=pltpu.CompilerParams(use_tc_tiling_on_sc=False),
    )
    def seg_sum(table, src, dst, out, idx_s, idx_d, *bufs):
        rows = bufs[:_B]
        zbuf = bufs[_B]
        acc = bufs[_B + 1]
        gs = bufs[_B + 2:_B + 2 + _F]
        ss = bufs[_B + 2 + _F:]
        cid = lax.axis_index("c")
        sid = lax.axis_index("s")
        wid = cid * _NS + sid

        zv = jnp.zeros((16,), _F32)

        def _z(i, _):
            zbuf[i, pl.ds(0, 16)] = zv
            return 0

        lax.fori_loop(0, zr, _z, 0)

        if not gather:
            ov = jnp.ones((16,), _F32)

            def _o(i, _):
                rows[0][i, pl.ds(0, 16)] = ov
                return 0

            lax.fori_loop(0, _CH, _o, 0)

        def _zero(i, _):
            ch = i * _NS + sid

            @pl.when(ch < nchunks)
            def _():
                pltpu.sync_copy(zbuf, acc.at[pl.ds(ch * zr, zr)])
            return 0

        lax.fori_loop(0, per_tile, _zero, 0)
        plsc.subcore_barrier()

        if gather:
            pltpu.sync_copy(src.at[wid], idx_s)
        pltpu.sync_copy(dst.at[wid], idx_d)

        if gather:
            def _one(c, p, wait_sct, issue_gat):
                s = p % _F
                pltpu.make_async_copy(
                    table.at[idx_s.at[c]], rows[p], gs[s]).wait()
                if wait_sct:
                    pltpu.make_async_copy(
                        rows[(p + _B - _F) % _B], acc.at[idx_d.at[c - _F]],
                        ss[s]).wait()
                pltpu.async_copy(rows[p], acc.at[idx_d.at[c]], ss[s], add=True)
                if issue_gat:
                    pltpu.async_copy(
                        table.at[idx_s.at[c + _F]], rows[(p + _F) % _B], gs[s])

            for c in range(min(_F, k_chunks)):
                pltpu.async_copy(table.at[idx_s.at[c]], rows[c % _B], gs[c % _F])
            bmax = max((k_chunks - _F) // _B, 1) if k_chunks > _B else 1
            for c in range(min(_B, k_chunks)):
                _one(c, c, c >= _F, c + _F < k_chunks)

            def _blk(bi, _):
                for p in range(_B):
                    _one(bi * _B + p, p, True, True)
                return 0

            if bmax > 1:
                lax.fori_loop(1, bmax, _blk, 0)
            for c in range(max(bmax * _B, min(_B, k_chunks)), k_chunks):
                _one(c, c % _B, True, c + _F < k_chunks)
            for m in range(max(0, k_chunks - _F), k_chunks):
                pltpu.make_async_copy(
                    rows[m % _B], acc.at[idx_d.at[m]], ss[m % _F]).wait()
        else:
            for j in range(min(_F, k_chunks)):
                pltpu.async_copy(rows[0], acc.at[idx_d.at[j]], ss[j % _F],
                                 add=True)

            def _cblk(bi, _):
                for q in range(_F):
                    j = bi * _F + q
                    pltpu.make_async_copy(
                        rows[0], acc.at[idx_d.at[j - _F]], ss[q]).wait()
                    pltpu.async_copy(rows[0], acc.at[idx_d.at[j]], ss[q],
                                     add=True)
                return 0

            cmax = k_chunks // _F
            if cmax > 1:
                lax.fori_loop(1, cmax, _cblk, 0)
            for j in range(cmax * _F, k_chunks):
                pltpu.make_async_copy(
                    rows[0], acc.at[idx_d.at[j - _F]], ss[j % _F]).wait()
                pltpu.async_copy(rows[0], acc.at[idx_d.at[j]], ss[j % _F],
                                 add=True)
            for m in range(max(0, k_chunks - _F), k_chunks):
                pltpu.make_async_copy(
                    rows[0], acc.at[idx_d.at[m]], ss[m % _F]).wait()
        plsc.subcore_barrier()

        def _drain(i, _):
            ch = i * _NS + sid

            @pl.when(ch < nchunks)
            def _():
                r0 = ch * zr
                pltpu.sync_copy(acc.at[pl.ds(r0, zr)], out.at[cid, pl.ds(r0, zr)])
            return 0

        lax.fori_loop(0, per_tile, _drain, 0)

    return seg_sum


def _full(shape):
    return pl.BlockSpec(shape, lambda i: tuple(0 for _ in shape))


def _row(blk, w):
    return pl.BlockSpec((blk, w), lambda i: (i, 0))


def _agg(blk):
    return pl.BlockSpec((2, blk, _W), lambda i: (0, i, 0))


@functools.lru_cache(maxsize=None)
def _make_tc1(n, blk):
    def body(x_ref, wl_ref, wr_ref, b_ref, y_ref, z_ref):
        xb = x_ref[...]
        y_ref[...] = jnp.dot(xb, wl_ref[...], preferred_element_type=_F32)
        z_ref[...] = jnp.dot(xb, wr_ref[...], preferred_element_type=_F32) + b_ref[...]

    return pl.pallas_call(
        body,
        grid=(n // blk,),
        in_specs=[_row(blk, 56), _full((56, 16)), _full((56, 16)), _full((1, 16))],
        out_specs=[_row(blk, 16), _row(blk, 16)],
        out_shape=[jax.ShapeDtypeStruct((n, 16), _F32),
                   jax.ShapeDtypeStruct((n, 16), _F32)],
    )


@functools.lru_cache(maxsize=None)
def _make_tc2(n, blk):
    def body(a_ref, c_ref, z_ref, h_ref, inv_ref):
        a = a_ref[0] + a_ref[1]
        cnt = c_ref[0, :, 0:1] + c_ref[1, :, 0:1]
        inv = 1.0 / jnp.maximum(cnt, 1.0)
        h_ref[...] = jnp.maximum(a * inv + z_ref[...], 0.0)
        inv_ref[...] = inv

    return pl.pallas_call(
        body,
        grid=(n // blk,),
        in_specs=[_agg(blk), _agg(blk), _row(blk, 16)],
        out_specs=[_row(blk, 16), _row(blk, 1)],
        out_shape=[jax.ShapeDtypeStruct((n, 16), _F32),
                   jax.ShapeDtypeStruct((n, 1), _F32)],
    )


@functools.lru_cache(maxsize=None)
def _make_tc3(n, blk):
    def body(a_ref, inv_ref, h_ref, wl_ref, wr_ref, b_ref, oa_ref, ob_ref):
        mean = (a_ref[0] + a_ref[1]) * inv_ref[...]
        o = jnp.maximum(
            jnp.dot(mean, wl_ref[...], preferred_element_type=_F32)
            + jnp.dot(h_ref[...], wr_ref[...], preferred_element_type=_F32)
            + b_ref[...], 0.0)
        oa_ref[...] = o[:, :16]
        ob_ref[...] = o[:, 16:]

    return pl.pallas_call(
        body,
        grid=(n // blk,),
        in_specs=[_agg(blk), _row(blk, 1), _row(blk, 16),
                  _full((16, 32)), _full((16, 32)), _full((1, 32))],
        out_specs=[_row(blk, 16), _row(blk, 16)],
        out_shape=[jax.ShapeDtypeStruct((n, 16), _F32),
                   jax.ShapeDtypeStruct((n, 16), _F32)],
    )


@functools.lru_cache(maxsize=None)
def _make_tc4(n, blk):
    def body(aa_ref, ab_ref, inv_ref, ha_ref, hb_ref, wl3_ref, wr3_ref,
             b3_ref, wl4_ref, wr4_ref, b4_ref, y_ref, z_ref):
        inv = inv_ref[...]
        mean = jnp.concatenate(
            [(aa_ref[0] + aa_ref[1]) * inv, (ab_ref[0] + ab_ref[1]) * inv],
            axis=1)
        h2 = jnp.concatenate([ha_ref[...], hb_ref[...]], axis=1)
        h3 = jnp.maximum(
            jnp.dot(mean, wl3_ref[...], preferred_element_type=_F32)
            + jnp.dot(h2, wr3_ref[...], preferred_element_type=_F32)
            + b3_ref[...], 0.0)
        pad = jnp.zeros((blk, 14), _F32)
        y_ref[...] = jnp.concatenate(
            [jnp.dot(h3, wl4_ref[...], preferred_element_type=_F32), pad], axis=1)
        z_ref[...] = jnp.concatenate(
            [jnp.dot(h3, wr4_ref[...], preferred_element_type=_F32) + b4_ref[...],
             pad], axis=1)

    return pl.pallas_call(
        body,
        grid=(n // blk,),
        in_specs=[_agg(blk), _agg(blk), _row(blk, 1), _row(blk, 16),
                  _row(blk, 16), _full((32, 64)), _full((32, 64)),
                  _full((1, 64)), _full((64, 2)), _full((64, 2)), _full((1, 2))],
        out_specs=[_row(blk, 16), _row(blk, 16)],
        out_shape=[jax.ShapeDtypeStruct((n, 16), _F32),
                   jax.ShapeDtypeStruct((n, 16), _F32)],
    )


@functools.lru_cache(maxsize=None)
def _make_tc5(n, blk):
    nb = n // blk

    def body(a_ref, inv_ref, z4_ref, bt_ref, o_ref, acc_ref):
        i = pl.program_id(0)
        a = a_ref[0] + a_ref[1]
        h4 = jnp.maximum(a[:, :2] * inv_ref[...] + z4_ref[:, :2], 0.0)
        hc = jnp.concatenate(
            [h4, jnp.ones((blk, 1), _F32), jnp.zeros((blk, 5), _F32)], axis=1)
        bt = bt_ref[...][0, 0]
        oh = (bt[:, None] == lax.broadcasted_iota(jnp.int32, (blk, _G), 1)
              ).astype(_F32)
        part = lax.dot_general(oh, hc, (((0,), (0,)), ((), ())),
                               preferred_element_type=_F32)

        @pl.when(i == 0)
        def _():
            acc_ref[...] = part

        @pl.when(i > 0)
        def _():
            acc_ref[...] = acc_ref[...] + part

        @pl.when(i == nb - 1)
        def _():
            acc = acc_ref[...]
            pooled = acc[:, :2] / jnp.maximum(acc[:, 2:3], 1.0)
            m = jnp.max(pooled, axis=1, keepdims=True)
            e = jnp.exp(pooled - m)
            o_ref[...] = e / jnp.sum(e, axis=1, keepdims=True)

    return pl.pallas_call(
        body,
        grid=(nb,),
        in_specs=[_agg(blk), _row(blk, 1), _row(blk, 16),
                  pl.BlockSpec((1, 1, blk), lambda i: (i, 0, 0))],
        out_specs=pl.BlockSpec((_G, 2), lambda i: (0, 0)),
        out_shape=jax.ShapeDtypeStruct((_G, 2), _F32),
        scratch_shapes=[pltpu.VMEM((_G, 8), _F32)],
    )


def kernel(x, edge_index, batch, Wl1, Wr1, b1, Wl2, Wr2, b2,
           Wl3, Wr3, b3, Wl4, Wr4, b4):
    n = x.shape[0]
    e = edge_index.shape[1]
    blk = next(d for d in range(min(2048, n), 7, -1) if n % d == 0 and d % 8 == 0)
    nb = n // blk

    k = -(-e // (_NW * _CH))
    ep = _NW * k * _CH
    src = jnp.concatenate(
        [edge_index[0], jnp.zeros((ep - e,), jnp.int32)]).reshape(_NW, k, _CH)
    dst = jnp.concatenate(
        [edge_index[1], jnp.full((ep - e,), n, jnp.int32)]).reshape(_NW, k, _CH)

    seg = _make_seg_sum(n, k, True)
    cnt = _make_seg_sum(n, k, False)

    y1, z1 = _make_tc1(n, blk)(x, Wl1, Wr1, b1.reshape(1, -1))
    c = cnt(y1, src, dst)
    a1 = seg(y1, src, dst)
    h1, inv = _make_tc2(n, blk)(a1, c, z1)
    a2 = seg(h1, src, dst)
    h2a, h2b = _make_tc3(n, blk)(a2, inv, h1, Wl2, Wr2, b2.reshape(1, -1))
    a3a = seg(h2a, src, dst)
    a3b = seg(h2b, src, dst)
    y4, z4 = _make_tc4(n, blk)(a3a, a3b, inv, h2a, h2b, Wl3, Wr3,
                               b3.reshape(1, -1), Wl4, Wr4, b4.reshape(1, -1))
    a4 = seg(y4, src, dst)
    return _make_tc5(n, blk)(a4, inv, z4, batch.reshape(nb, 1, blk))


# speedup vs baseline: 16.7738x; 1.0453x over previous
import functools

import jax
import jax.numpy as jnp
from jax import lax
from jax.experimental import pallas as pl
from jax.experimental.pallas import tpu as pltpu
from jax.experimental.pallas import tpu_sc as plsc

_F32 = jnp.float32
_NC, _NS = 2, 16
_NW = _NC * _NS
_CH = 128
_W = 16
_B = 12
_F = 6
_G = 64


@functools.lru_cache(maxsize=None)
def _make_seg_sum(n_nodes, k_chunks, gather):
    zr = next(d for d in range(min(256, n_nodes), 7, -1)
              if n_nodes % d == 0 and d % 8 == 0)
    nchunks = n_nodes // zr
    per_tile = -(-nchunks // _NS)
    n_acc = n_nodes + 8
    mesh = plsc.VectorSubcoreMesh(core_axis_name="c", subcore_axis_name="s")

    @functools.partial(
        pl.kernel,
        out_type=jax.ShapeDtypeStruct((_NC, n_nodes, _W), _F32),
        mesh=mesh,
        scratch_types=[
            pltpu.VMEM((k_chunks, _CH), jnp.int32),
            pltpu.VMEM((k_chunks, _CH), jnp.int32),
        ] + [pltpu.VMEM((_CH, _W), _F32)] * _B + [
            pltpu.VMEM((zr, _W), _F32),
            pltpu.VMEM_SHARED((n_acc, _W), _F32),
        ] + [pltpu.SemaphoreType.DMA] * (2 * _F),
        compiler_params=pltpu.CompilerParams(use_tc_tiling_on_sc=False),
    )
    def seg_sum(table, src, dst, out, idx_s, idx_d, *bufs):
        rows = bufs[:_B]
        zbuf = bufs[_B]
        acc = bufs[_B + 1]
        gs = bufs[_B + 2:_B + 2 + _F]
        ss = bufs[_B + 2 + _F:]
        cid = lax.axis_index("c")
        sid = lax.axis_index("s")
        wid = cid * _NS + sid

        zv = jnp.zeros((16,), _F32)

        def _z(i, _):
            zbuf[i, pl.ds(0, 16)] = zv
            return 0

        lax.fori_loop(0, zr, _z, 0)

        if not gather:
            ov = jnp.ones((16,), _F32)

            def _o(i, _):
                rows[0][i, pl.ds(0, 16)] = ov
                return 0

            lax.fori_loop(0, _CH, _o, 0)

        def _zero(i, _):
            ch = i * _NS + sid

            @pl.when(ch < nchunks)
            def _():
                pltpu.sync_copy(zbuf, acc.at[pl.ds(ch * zr, zr)])
            return 0

        lax.fori_loop(0, per_tile, _zero, 0)
        plsc.subcore_barrier()

        if gather:
            pltpu.sync_copy(src.at[wid], idx_s)
        pltpu.sync_copy(dst.at[wid], idx_d)

        if gather:
            def _one(c, p, wait_sct, issue_gat):
                s = p % _F
                pltpu.make_async_copy(
                    table.at[idx_s.at[c]], rows[p], gs[s]).wait()
                if wait_sct:
                    pltpu.make_async_copy(
                        rows[(p + _B - _F) % _B], acc.at[idx_d.at[c - _F]],
                        ss[s]).wait()
                pltpu.async_copy(rows[p], acc.at[idx_d.at[c]], ss[s], add=True)
                if issue_gat:
                    pltpu.async_copy(
                        table.at[idx_s.at[c + _F]], rows[(p + _F) % _B], gs[s])

            for c in range(min(_F, k_chunks)):
                pltpu.async_copy(table.at[idx_s.at[c]], rows[c % _B], gs[c % _F])
            bmax = max((k_chunks - _F) // _B, 1) if k_chunks > _B else 1
            for c in range(min(_B, k_chunks)):
                _one(c, c, c >= _F, c + _F < k_chunks)

            def _blk(bi, _):
                for p in range(_B):
                    _one(bi * _B + p, p, True, True)
                return 0

            if bmax > 1:
                lax.fori_loop(1, bmax, _blk, 0)
            for c in range(max(bmax * _B, min(_B, k_chunks)), k_chunks):
                _one(c, c % _B, True, c + _F < k_chunks)
            for m in range(max(0, k_chunks - _F), k_chunks):
                pltpu.make_async_copy(
                    rows[m % _B], acc.at[idx_d.at[m]], ss[m % _F]).wait()
        else:
            for j in range(min(_F, k_chunks)):
                pltpu.async_copy(rows[0], acc.at[idx_d.at[j]], ss[j % _F],
                                 add=True)

            def _cblk(bi, _):
                for q in range(_F):
                    j = bi * _F + q
                    pltpu.make_async_copy(
                        rows[0], acc.at[idx_d.at[j - _F]], ss[q]).wait()
                    pltpu.async_copy(rows[0], acc.at[idx_d.at[j]], ss[q],
                                     add=True)
                return 0

            cmax = k_chunks // _F
            if cmax > 1:
                lax.fori_loop(1, cmax, _cblk, 0)
            for j in range(cmax * _F, k_chunks):
                pltpu.make_async_copy(
                    rows[0], acc.at[idx_d.at[j - _F]], ss[j % _F]).wait()
                pltpu.async_copy(rows[0], acc.at[idx_d.at[j]], ss[j % _F],
                                 add=True)
            for m in range(max(0, k_chunks - _F), k_chunks):
                pltpu.make_async_copy(
                    rows[0], acc.at[idx_d.at[m]], ss[m % _F]).wait()
        plsc.subcore_barrier()

        def _drain(i, _):
            ch = i * _NS + sid

            @pl.when(ch < nchunks)
            def _():
                r0 = ch * zr
                pltpu.sync_copy(acc.at[pl.ds(r0, zr)], out.at[cid, pl.ds(r0, zr)])
            return 0

        lax.fori_loop(0, per_tile, _drain, 0)

    return seg_sum


def _full(shape):
    return pl.BlockSpec(shape, lambda i: tuple(0 for _ in shape))


def _row(blk, w):
    return pl.BlockSpec((blk, w), lambda i: (i, 0))


def _agg(blk):
    return pl.BlockSpec((2, blk, _W), lambda i: (0, i, 0))


@functools.lru_cache(maxsize=None)
def _make_tc1(n, blk):
    def body(x_ref, wl_ref, wr_ref, b_ref, y_ref, z_ref):
        xb = x_ref[...]
        y_ref[...] = jnp.dot(xb, wl_ref[...], preferred_element_type=_F32)
        z_ref[...] = jnp.dot(xb, wr_ref[...], preferred_element_type=_F32) + b_ref[...]

    return pl.pallas_call(
        body,
        grid=(n // blk,),
        in_specs=[_row(blk, 56), _full((56, 16)), _full((56, 16)), _full((1, 16))],
        out_specs=[_row(blk, 16), _row(blk, 16)],
        out_shape=[jax.ShapeDtypeStruct((n, 16), _F32),
                   jax.ShapeDtypeStruct((n, 16), _F32)],
    )


@functools.lru_cache(maxsize=None)
def _make_tc2(n, blk):
    def body(a_ref, c_ref, z_ref, h_ref, inv_ref):
        a = a_ref[0] + a_ref[1]
        cnt = c_ref[0, :, 0:1] + c_ref[1, :, 0:1]
        inv = 1.0 / jnp.maximum(cnt, 1.0)
        h_ref[...] = jnp.maximum(a * inv + z_ref[...], 0.0)
        inv_ref[...] = inv

    return pl.pallas_call(
        body,
        grid=(n // blk,),
        in_specs=[_agg(blk), _agg(blk), _row(blk, 16)],
        out_specs=[_row(blk, 16), _row(blk, 1)],
        out_shape=[jax.ShapeDtypeStruct((n, 16), _F32),
                   jax.ShapeDtypeStruct((n, 1), _F32)],
    )


@functools.lru_cache(maxsize=None)
def _make_tc3(n, blk):
    def body(a_ref, inv_ref, h_ref, wl_ref, wr_ref, b_ref, oa_ref, ob_ref):
        mean = (a_ref[0] + a_ref[1]) * inv_ref[...]
        o = jnp.maximum(
            jnp.dot(mean, wl_ref[...], preferred_element_type=_F32)
            + jnp.dot(h_ref[...], wr_ref[...], preferred_element_type=_F32)
            + b_ref[...], 0.0)
        oa_ref[...] = o[:, :16]
        ob_ref[...] = o[:, 16:]

    return pl.pallas_call(
        body,
        grid=(n // blk,),
        in_specs=[_agg(blk), _row(blk, 1), _row(blk, 16),
                  _full((16, 32)), _full((16, 32)), _full((1, 32))],
        out_specs=[_row(blk, 16), _row(blk, 16)],
        out_shape=[jax.ShapeDtypeStruct((n, 16), _F32),
                   jax.ShapeDtypeStruct((n, 16), _F32)],
    )


@functools.lru_cache(maxsize=None)
def _make_tc4(n, blk):
    def body(aa_ref, ab_ref, inv_ref, ha_ref, hb_ref, wl3_ref, wr3_ref,
             b3_ref, wl4_ref, wr4_ref, b4_ref, y_ref, z_ref):
        inv = inv_ref[...]
        mean = jnp.concatenate(
            [(aa_ref[0] + aa_ref[1]) * inv, (ab_ref[0] + ab_ref[1]) * inv],
            axis=1)
        h2 = jnp.concatenate([ha_ref[...], hb_ref[...]], axis=1)
        h3 = jnp.maximum(
            jnp.dot(mean, wl3_ref[...], preferred_element_type=_F32)
            + jnp.dot(h2, wr3_ref[...], preferred_element_type=_F32)
            + b3_ref[...], 0.0)
        pad = jnp.zeros((blk, 14), _F32)
        y_ref[...] = jnp.concatenate(
            [jnp.dot(h3, wl4_ref[...], preferred_element_type=_F32), pad], axis=1)
        z_ref[...] = jnp.concatenate(
            [jnp.dot(h3, wr4_ref[...], preferred_element_type=_F32) + b4_ref[...],
             pad], axis=1)

    return pl.pallas_call(
        body,
        grid=(n // blk,),
        in_specs=[_agg(blk), _agg(blk), _row(blk, 1), _row(blk, 16),
                  _row(blk, 16), _full((32, 64)), _full((32, 64)),
                  _full((1, 64)), _full((64, 2)), _full((64, 2)), _full((1, 2))],
        out_specs=[_row(blk, 16), _row(blk, 16)],
        out_shape=[jax.ShapeDtypeStruct((n, 16), _F32),
                   jax.ShapeDtypeStruct((n, 16), _F32)],
    )


@functools.lru_cache(maxsize=None)
def _make_tc5(n, blk):
    nb = n // blk

    def body(a_ref, inv_ref, z4_ref, bt_ref, o_ref, acc_ref):
        i = pl.program_id(0)
        a = a_ref[0] + a_ref[1]
        h4 = jnp.maximum(a[:, :2] * inv_ref[...] + z4_ref[:, :2], 0.0)
        hc = jnp.concatenate(
            [h4, jnp.ones((blk, 1), _F32), jnp.zeros((blk, 5), _F32)], axis=1)
        bt = bt_ref[...][0, 0]
        oh = (bt[:, None] == lax.broadcasted_iota(jnp.int32, (blk, _G), 1)
              ).astype(_F32)
        part = lax.dot_general(oh, hc, (((0,), (0,)), ((), ())),
                               preferred_element_type=_F32)

        @pl.when(i == 0)
        def _():
            acc_ref[...] = part

        @pl.when(i > 0)
        def _():
            acc_ref[...] = acc_ref[...] + part

        @pl.when(i == nb - 1)
        def _():
            acc = acc_ref[...]
            pooled = acc[:, :2] / jnp.maximum(acc[:, 2:3], 1.0)
            m = jnp.max(pooled, axis=1, keepdims=True)
            e = jnp.exp(pooled - m)
            o_ref[...] = e / jnp.sum(e, axis=1, keepdims=True)

    return pl.pallas_call(
        body,
        grid=(nb,),
        in_specs=[_agg(blk), _row(blk, 1), _row(blk, 16),
                  pl.BlockSpec((1, 1, blk), lambda i: (i, 0, 0))],
        out_specs=pl.BlockSpec((_G, 2), lambda i: (0, 0)),
        out_shape=jax.ShapeDtypeStruct((_G, 2), _F32),
        scratch_shapes=[pltpu.VMEM((_G, 8), _F32)],
    )


def kernel(x, edge_index, batch, Wl1, Wr1, b1, Wl2, Wr2, b2,
           Wl3, Wr3, b3, Wl4, Wr4, b4):
    n = x.shape[0]
    e = edge_index.shape[1]
    blk = next(d for d in range(min(2048, n), 7, -1) if n % d == 0 and d % 8 == 0)
    nb = n // blk

    k = -(-e // (_NW * _CH))
    ep = _NW * k * _CH
    src = jnp.concatenate(
        [edge_index[0], jnp.zeros((ep - e,), jnp.int32)]).reshape(_NW, k, _CH)
    dst = jnp.concatenate(
        [edge_index[1], jnp.full((ep - e,), n, jnp.int32)]).reshape(_NW, k, _CH)

    seg = _make_seg_sum(n, k, True)
    cnt = _make_seg_sum(n, k, False)

    y1, z1 = _make_tc1(n, blk)(x, Wl1, Wr1, b1.reshape(1, -1))
    c = cnt(y1, src, dst)
    a1 = seg(y1, src, dst)
    h1, inv = _make_tc2(n, blk)(a1, c, z1)
    a2 = seg(h1, src, dst)
    h2a, h2b = _make_tc3(n, blk)(a2, inv, h1, Wl2, Wr2, b2.reshape(1, -1))
    a3a = seg(h2a, src, dst)
    a3b = seg(h2b, src, dst)
    y4, z4 = _make_tc4(n, blk)(a3a, a3b, inv, h2a, h2b, Wl3, Wr3,
                               b3.reshape(1, -1), Wl4, Wr4, b4.reshape(1, -1))
    a4 = seg(y4, src, dst)
    return _make_tc5(n, blk)(a4, inv, z4, batch.reshape(nb, 1, blk))


# speedup vs baseline: 17.4452x; 1.0400x over previous
import functools

import jax
import jax.numpy as jnp
from jax import lax
from jax.experimental import pallas as pl
from jax.experimental.pallas import tpu as pltpu
from jax.experimental.pallas import tpu_sc as plsc

_F32 = jnp.float32
_NC, _NS = 2, 16
_NW = _NC * _NS
_CH = 128
_W = 16
_B = 12
_F = 6
_G = 64


@functools.lru_cache(maxsize=None)
def _make_seg_sum(n_nodes, k_chunks, gather):
    zr = next(d for d in range(min(256, n_nodes), 7, -1)
              if n_nodes % d == 0 and d % 8 == 0)
    nchunks = n_nodes // zr
    per_tile = -(-nchunks // _NS)
    n_acc = n_nodes + 8
    mesh = plsc.VectorSubcoreMesh(core_axis_name="c", subcore_axis_name="s")

    @functools.partial(
        pl.kernel,
        out_type=jax.ShapeDtypeStruct((_NC, n_nodes, _W), _F32),
        mesh=mesh,
        scratch_types=[
            pltpu.VMEM((k_chunks, _CH), jnp.int32),
            pltpu.VMEM((k_chunks, _CH), jnp.int32),
        ] + [pltpu.VMEM((_CH, _W), _F32)] * _B + [
            pltpu.VMEM((zr, _W), _F32),
            pltpu.VMEM_SHARED((n_acc, _W), _F32),
        ] + [pltpu.SemaphoreType.DMA] * (2 * _F),
        compiler_params=pltpu.CompilerParams(use_tc_tiling_on_sc=False),
    )
    def seg_sum(table, src, dst, out, idx_s, idx_d, *bufs):
        rows = bufs[:_B]
        zbuf = bufs[_B]
        acc = bufs[_B + 1]
        gs = bufs[_B + 2:_B + 2 + _F]
        ss = bufs[_B + 2 + _F:]
        cid = lax.axis_index("c")
        sid = lax.axis_index("s")
        wid = cid * _NS + sid

        zv = jnp.zeros((16,), _F32)

        def _z(i, _):
            zbuf[i, pl.ds(0, 16)] = zv
            return 0

        lax.fori_loop(0, zr, _z, 0)

        if not gather:
            ov = jnp.ones((16,), _F32)

            def _o(i, _):
                rows[0][i, pl.ds(0, 16)] = ov
                return 0

            lax.fori_loop(0, _CH, _o, 0)

        for i in range(per_tile):
            if i >= _F:
                cw = (i - _F) * _NS + sid

                @pl.when(cw < nchunks)
                def _(cw=cw, s=(i - _F) % _F):
                    pltpu.make_async_copy(
                        zbuf, acc.at[pl.ds(cw * zr, zr)], gs[s]).wait()
            ch = i * _NS + sid

            @pl.when(ch < nchunks)
            def _(ch=ch, s=i % _F):
                pltpu.async_copy(zbuf, acc.at[pl.ds(ch * zr, zr)], gs[s])
        for m in range(max(0, per_tile - _F), per_tile):
            cm = m * _NS + sid

            @pl.when(cm < nchunks)
            def _(cm=cm, s=m % _F):
                pltpu.make_async_copy(
                    zbuf, acc.at[pl.ds(cm * zr, zr)], gs[s]).wait()
        plsc.subcore_barrier()

        if gather:
            pltpu.sync_copy(src.at[wid], idx_s)
        pltpu.sync_copy(dst.at[wid], idx_d)

        if gather:
            def _one(c, p, wait_sct, issue_gat):
                s = p % _F
                pltpu.make_async_copy(
                    table.at[idx_s.at[c]], rows[p], gs[s]).wait()
                if wait_sct:
                    pltpu.make_async_copy(
                        rows[(p + _B - _F) % _B], acc.at[idx_d.at[c - _F]],
                        ss[s]).wait()
                pltpu.async_copy(rows[p], acc.at[idx_d.at[c]], ss[s], add=True)
                if issue_gat:
                    pltpu.async_copy(
                        table.at[idx_s.at[c + _F]], rows[(p + _F) % _B], gs[s])

            for c in range(min(_F, k_chunks)):
                pltpu.async_copy(table.at[idx_s.at[c]], rows[c % _B], gs[c % _F])
            bmax = max((k_chunks - _F) // _B, 1) if k_chunks > _B else 1
            for c in range(min(_B, k_chunks)):
                _one(c, c, c >= _F, c + _F < k_chunks)

            def _blk(bi, _):
                for p in range(_B):
                    _one(bi * _B + p, p, True, True)
                return 0

            if bmax > 1:
                lax.fori_loop(1, bmax, _blk, 0)
            for c in range(max(bmax * _B, min(_B, k_chunks)), k_chunks):
                _one(c, c % _B, True, c + _F < k_chunks)
            for m in range(max(0, k_chunks - _F), k_chunks):
                pltpu.make_async_copy(
                    rows[m % _B], acc.at[idx_d.at[m]], ss[m % _F]).wait()
        else:
            for j in range(min(_F, k_chunks)):
                pltpu.async_copy(rows[0], acc.at[idx_d.at[j]], ss[j % _F],
                                 add=True)

            def _cblk(bi, _):
                for q in range(_F):
                    j = bi * _F + q
                    pltpu.make_async_copy(
                        rows[0], acc.at[idx_d.at[j - _F]], ss[q]).wait()
                    pltpu.async_copy(rows[0], acc.at[idx_d.at[j]], ss[q],
                                     add=True)
                return 0

            cmax = k_chunks // _F
            if cmax > 1:
                lax.fori_loop(1, cmax, _cblk, 0)
            for j in range(cmax * _F, k_chunks):
                pltpu.make_async_copy(
                    rows[0], acc.at[idx_d.at[j - _F]], ss[j % _F]).wait()
                pltpu.async_copy(rows[0], acc.at[idx_d.at[j]], ss[j % _F],
                                 add=True)
            for m in range(max(0, k_chunks - _F), k_chunks):
                pltpu.make_async_copy(
                    rows[0], acc.at[idx_d.at[m]], ss[m % _F]).wait()
        plsc.subcore_barrier()

        for i in range(per_tile):
            if i >= _F:
                cw = (i - _F) * _NS + sid

                @pl.when(cw < nchunks)
                def _(cw=cw, s=(i - _F) % _F):
                    r0 = cw * zr
                    pltpu.make_async_copy(
                        acc.at[pl.ds(r0, zr)], out.at[cid, pl.ds(r0, zr)],
                        gs[s]).wait()
            ch = i * _NS + sid

            @pl.when(ch < nchunks)
            def _(ch=ch, s=i % _F):
                r0 = ch * zr
                pltpu.async_copy(
                    acc.at[pl.ds(r0, zr)], out.at[cid, pl.ds(r0, zr)], gs[s])
        for m in range(max(0, per_tile - _F), per_tile):
            cm = m * _NS + sid

            @pl.when(cm < nchunks)
            def _(cm=cm, s=m % _F):
                r0 = cm * zr
                pltpu.make_async_copy(
                    acc.at[pl.ds(r0, zr)], out.at[cid, pl.ds(r0, zr)],
                    gs[s]).wait()

    return seg_sum


def _full(shape):
    return pl.BlockSpec(shape, lambda i: tuple(0 for _ in shape))


def _row(blk, w):
    return pl.BlockSpec((blk, w), lambda i: (i, 0))


def _agg(blk):
    return pl.BlockSpec((2, blk, _W), lambda i: (0, i, 0))


@functools.lru_cache(maxsize=None)
def _make_tc1(n, blk):
    def body(x_ref, wl_ref, wr_ref, b_ref, y_ref, z_ref):
        xb = x_ref[...]
        y_ref[...] = jnp.dot(xb, wl_ref[...], preferred_element_type=_F32)
        z_ref[...] = jnp.dot(xb, wr_ref[...], preferred_element_type=_F32) + b_ref[...]

    return pl.pallas_call(
        body,
        grid=(n // blk,),
        in_specs=[_row(blk, 56), _full((56, 16)), _full((56, 16)), _full((1, 16))],
        out_specs=[_row(blk, 16), _row(blk, 16)],
        out_shape=[jax.ShapeDtypeStruct((n, 16), _F32),
                   jax.ShapeDtypeStruct((n, 16), _F32)],
    )


@functools.lru_cache(maxsize=None)
def _make_tc2(n, blk):
    def body(a_ref, c_ref, z_ref, h_ref, inv_ref):
        a = a_ref[0] + a_ref[1]
        cnt = c_ref[0, :, 0:1] + c_ref[1, :, 0:1]
        inv = 1.0 / jnp.maximum(cnt, 1.0)
        h_ref[...] = jnp.maximum(a * inv + z_ref[...], 0.0)
        inv_ref[...] = inv

    return pl.pallas_call(
        body,
        grid=(n // blk,),
        in_specs=[_agg(blk), _agg(blk), _row(blk, 16)],
        out_specs=[_row(blk, 16), _row(blk, 1)],
        out_shape=[jax.ShapeDtypeStruct((n, 16), _F32),
                   jax.ShapeDtypeStruct((n, 1), _F32)],
    )


@functools.lru_cache(maxsize=None)
def _make_tc3(n, blk):
    def body(a_ref, inv_ref, h_ref, wl_ref, wr_ref, b_ref, oa_ref, ob_ref):
        mean = (a_ref[0] + a_ref[1]) * inv_ref[...]
        o = jnp.maximum(
            jnp.dot(mean, wl_ref[...], preferred_element_type=_F32)
            + jnp.dot(h_ref[...], wr_ref[...], preferred_element_type=_F32)
            + b_ref[...], 0.0)
        oa_ref[...] = o[:, :16]
        ob_ref[...] = o[:, 16:]

    return pl.pallas_call(
        body,
        grid=(n // blk,),
        in_specs=[_agg(blk), _row(blk, 1), _row(blk, 16),
                  _full((16, 32)), _full((16, 32)), _full((1, 32))],
        out_specs=[_row(blk, 16), _row(blk, 16)],
        out_shape=[jax.ShapeDtypeStruct((n, 16), _F32),
                   jax.ShapeDtypeStruct((n, 16), _F32)],
    )


@functools.lru_cache(maxsize=None)
def _make_tc4(n, blk):
    def body(aa_ref, ab_ref, inv_ref, ha_ref, hb_ref, wl3_ref, wr3_ref,
             b3_ref, wl4_ref, wr4_ref, b4_ref, y_ref, z_ref):
        inv = inv_ref[...]
        mean = jnp.concatenate(
            [(aa_ref[0] + aa_ref[1]) * inv, (ab_ref[0] + ab_ref[1]) * inv],
            axis=1)
        h2 = jnp.concatenate([ha_ref[...], hb_ref[...]], axis=1)
        h3 = jnp.maximum(
            jnp.dot(mean, wl3_ref[...], preferred_element_type=_F32)
            + jnp.dot(h2, wr3_ref[...], preferred_element_type=_F32)
            + b3_ref[...], 0.0)
        pad = jnp.zeros((blk, 14), _F32)
        y_ref[...] = jnp.concatenate(
            [jnp.dot(h3, wl4_ref[...], preferred_element_type=_F32), pad], axis=1)
        z_ref[...] = jnp.concatenate(
            [jnp.dot(h3, wr4_ref[...], preferred_element_type=_F32) + b4_ref[...],
             pad], axis=1)

    return pl.pallas_call(
        body,
        grid=(n // blk,),
        in_specs=[_agg(blk), _agg(blk), _row(blk, 1), _row(blk, 16),
                  _row(blk, 16), _full((32, 64)), _full((32, 64)),
                  _full((1, 64)), _full((64, 2)), _full((64, 2)), _full((1, 2))],
        out_specs=[_row(blk, 16), _row(blk, 16)],
        out_shape=[jax.ShapeDtypeStruct((n, 16), _F32),
                   jax.ShapeDtypeStruct((n, 16), _F32)],
    )


@functools.lru_cache(maxsize=None)
def _make_tc5(n, blk):
    nb = n // blk

    def body(a_ref, inv_ref, z4_ref, bt_ref, o_ref, acc_ref):
        i = pl.program_id(0)
        a = a_ref[0] + a_ref[1]
        h4 = jnp.maximum(a[:, :2] * inv_ref[...] + z4_ref[:, :2], 0.0)
        hc = jnp.concatenate(
            [h4, jnp.ones((blk, 1), _F32), jnp.zeros((blk, 5), _F32)], axis=1)
        bt = bt_ref[...][0, 0]
        oh = (bt[:, None] == lax.broadcasted_iota(jnp.int32, (blk, _G), 1)
              ).astype(_F32)
        part = lax.dot_general(oh, hc, (((0,), (0,)), ((), ())),
                               preferred_element_type=_F32)

        @pl.when(i == 0)
        def _():
            acc_ref[...] = part

        @pl.when(i > 0)
        def _():
            acc_ref[...] = acc_ref[...] + part

        @pl.when(i == nb - 1)
        def _():
            acc = acc_ref[...]
            pooled = acc[:, :2] / jnp.maximum(acc[:, 2:3], 1.0)
            m = jnp.max(pooled, axis=1, keepdims=True)
            e = jnp.exp(pooled - m)
            o_ref[...] = e / jnp.sum(e, axis=1, keepdims=True)

    return pl.pallas_call(
        body,
        grid=(nb,),
        in_specs=[_agg(blk), _row(blk, 1), _row(blk, 16),
                  pl.BlockSpec((1, 1, blk), lambda i: (i, 0, 0))],
        out_specs=pl.BlockSpec((_G, 2), lambda i: (0, 0)),
        out_shape=jax.ShapeDtypeStruct((_G, 2), _F32),
        scratch_shapes=[pltpu.VMEM((_G, 8), _F32)],
    )


def kernel(x, edge_index, batch, Wl1, Wr1, b1, Wl2, Wr2, b2,
           Wl3, Wr3, b3, Wl4, Wr4, b4):
    n = x.shape[0]
    e = edge_index.shape[1]
    blk = next(d for d in range(min(2048, n), 7, -1) if n % d == 0 and d % 8 == 0)
    nb = n // blk

    k = -(-e // (_NW * _CH))
    ep = _NW * k * _CH
    src = jnp.concatenate(
        [edge_index[0], jnp.zeros((ep - e,), jnp.int32)]).reshape(_NW, k, _CH)
    dst = jnp.concatenate(
        [edge_index[1], jnp.full((ep - e,), n, jnp.int32)]).reshape(_NW, k, _CH)

    seg = _make_seg_sum(n, k, True)
    cnt = _make_seg_sum(n, k, False)

    y1, z1 = _make_tc1(n, blk)(x, Wl1, Wr1, b1.reshape(1, -1))
    c = cnt(y1, src, dst)
    a1 = seg(y1, src, dst)
    h1, inv = _make_tc2(n, blk)(a1, c, z1)
    a2 = seg(h1, src, dst)
    h2a, h2b = _make_tc3(n, blk)(a2, inv, h1, Wl2, Wr2, b2.reshape(1, -1))
    a3a = seg(h2a, src, dst)
    a3b = seg(h2b, src, dst)
    y4, z4 = _make_tc4(n, blk)(a3a, a3b, inv, h2a, h2b, Wl3, Wr3,
                               b3.reshape(1, -1), Wl4, Wr4, b4.reshape(1, -1))
    a4 = seg(y4, src, dst)
    return _make_tc5(n, blk)(a4, inv, z4, batch.reshape(nb, 1, blk))


# speedup vs baseline: 17.7203x; 1.0158x over previous
import functools

import jax
import jax.numpy as jnp
from jax import lax
from jax.experimental import pallas as pl
from jax.experimental.pallas import tpu as pltpu
from jax.experimental.pallas import tpu_sc as plsc

_F32 = jnp.float32
_NC, _NS = 2, 16
_NW = _NC * _NS
_CH = 128
_W = 16
_B = 12
_F = 6
_G = 64


@functools.lru_cache(maxsize=None)
def _make_seg_sum(n_nodes, k_chunks, gather):
    zr = next(d for d in range(min(256, n_nodes), 7, -1)
              if n_nodes % d == 0 and d % 8 == 0)
    nchunks = n_nodes // zr
    per_tile = -(-nchunks // _NS)
    n_acc = n_nodes + 8
    mesh = plsc.VectorSubcoreMesh(core_axis_name="c", subcore_axis_name="s")

    @functools.partial(
        pl.kernel,
        out_type=jax.ShapeDtypeStruct((_NC, n_nodes, _W), _F32),
        mesh=mesh,
        scratch_types=[
            pltpu.VMEM((k_chunks, _CH), jnp.int32),
            pltpu.VMEM((k_chunks, _CH), jnp.int32),
        ] + [pltpu.VMEM((_CH, _W), _F32)] * _B + [
            pltpu.VMEM((zr, _W), _F32),
            pltpu.VMEM_SHARED((n_acc, _W), _F32),
        ] + [pltpu.SemaphoreType.DMA] * (2 * _F),
        compiler_params=pltpu.CompilerParams(use_tc_tiling_on_sc=False),
    )
    def seg_sum(table, src, dst, out, idx_s, idx_d, *bufs):
        rows = bufs[:_B]
        zbuf = bufs[_B]
        acc = bufs[_B + 1]
        gs = bufs[_B + 2:_B + 2 + _F]
        ss = bufs[_B + 2 + _F:]
        cid = lax.axis_index("c")
        sid = lax.axis_index("s")
        wid = cid * _NS + sid

        if gather:
            pltpu.async_copy(src.at[wid], idx_s, ss[0])
        pltpu.async_copy(dst.at[wid], idx_d, ss[1])

        zv = jnp.zeros((16,), _F32)

        def _z(i, _):
            zbuf[i, pl.ds(0, 16)] = zv
            return 0

        lax.fori_loop(0, zr, _z, 0)

        if not gather:
            ov = jnp.ones((16,), _F32)

            def _o(i, _):
                rows[0][i, pl.ds(0, 16)] = ov
                return 0

            lax.fori_loop(0, _CH, _o, 0)

        for i in range(per_tile):
            if i >= _F:
                cw = (i - _F) * _NS + sid

                @pl.when(cw < nchunks)
                def _(cw=cw, s=(i - _F) % _F):
                    pltpu.make_async_copy(
                        zbuf, acc.at[pl.ds(cw * zr, zr)], gs[s]).wait()
            ch = i * _NS + sid

            @pl.when(ch < nchunks)
            def _(ch=ch, s=i % _F):
                pltpu.async_copy(zbuf, acc.at[pl.ds(ch * zr, zr)], gs[s])
        for m in range(max(0, per_tile - _F), per_tile):
            cm = m * _NS + sid

            @pl.when(cm < nchunks)
            def _(cm=cm, s=m % _F):
                pltpu.make_async_copy(
                    zbuf, acc.at[pl.ds(cm * zr, zr)], gs[s]).wait()
        plsc.subcore_barrier()

        if gather:
            pltpu.make_async_copy(src.at[wid], idx_s, ss[0]).wait()
        pltpu.make_async_copy(dst.at[wid], idx_d, ss[1]).wait()

        if gather:
            def _one(c, p, wait_sct, issue_gat):
                s = p % _F
                pltpu.make_async_copy(
                    table.at[idx_s.at[c]], rows[p], gs[s]).wait()
                if wait_sct:
                    pltpu.make_async_copy(
                        rows[(p + _B - _F) % _B], acc.at[idx_d.at[c - _F]],
                        ss[s]).wait()
                pltpu.async_copy(rows[p], acc.at[idx_d.at[c]], ss[s], add=True)
                if issue_gat:
                    pltpu.async_copy(
                        table.at[idx_s.at[c + _F]], rows[(p + _F) % _B], gs[s])

            for c in range(min(_F, k_chunks)):
                pltpu.async_copy(table.at[idx_s.at[c]], rows[c % _B], gs[c % _F])
            bmax = max((k_chunks - _F) // _B, 1) if k_chunks > _B else 1
            for c in range(min(_B, k_chunks)):
                _one(c, c, c >= _F, c + _F < k_chunks)

            def _blk(bi, _):
                for p in range(_B):
                    _one(bi * _B + p, p, True, True)
                return 0

            if bmax > 1:
                lax.fori_loop(1, bmax, _blk, 0)
            for c in range(max(bmax * _B, min(_B, k_chunks)), k_chunks):
                _one(c, c % _B, True, c + _F < k_chunks)
            for m in range(max(0, k_chunks - _F), k_chunks):
                pltpu.make_async_copy(
                    rows[m % _B], acc.at[idx_d.at[m]], ss[m % _F]).wait()
        else:
            for j in range(min(_F, k_chunks)):
                pltpu.async_copy(rows[0], acc.at[idx_d.at[j]], ss[j % _F],
                                 add=True)

            def _cblk(bi, _):
                for q in range(_F):
                    j = bi * _F + q
                    pltpu.make_async_copy(
                        rows[0], acc.at[idx_d.at[j - _F]], ss[q]).wait()
                    pltpu.async_copy(rows[0], acc.at[idx_d.at[j]], ss[q],
                                     add=True)
                return 0

            cmax = k_chunks // _F
            if cmax > 1:
                lax.fori_loop(1, cmax, _cblk, 0)
            for j in range(cmax * _F, k_chunks):
                pltpu.make_async_copy(
                    rows[0], acc.at[idx_d.at[j - _F]], ss[j % _F]).wait()
                pltpu.async_copy(rows[0], acc.at[idx_d.at[j]], ss[j % _F],
                                 add=True)
            for m in range(max(0, k_chunks - _F), k_chunks):
                pltpu.make_async_copy(
                    rows[0], acc.at[idx_d.at[m]], ss[m % _F]).wait()
        plsc.subcore_barrier()

        for i in range(per_tile):
            if i >= _F:
                cw = (i - _F) * _NS + sid

                @pl.when(cw < nchunks)
                def _(cw=cw, s=(i - _F) % _F):
                    r0 = cw * zr
                    pltpu.make_async_copy(
                        acc.at[pl.ds(r0, zr)], out.at[cid, pl.ds(r0, zr)],
                        gs[s]).wait()
            ch = i * _NS + sid

            @pl.when(ch < nchunks)
            def _(ch=ch, s=i % _F):
                r0 = ch * zr
                pltpu.async_copy(
                    acc.at[pl.ds(r0, zr)], out.at[cid, pl.ds(r0, zr)], gs[s])
        for m in range(max(0, per_tile - _F), per_tile):
            cm = m * _NS + sid

            @pl.when(cm < nchunks)
            def _(cm=cm, s=m % _F):
                r0 = cm * zr
                pltpu.make_async_copy(
                    acc.at[pl.ds(r0, zr)], out.at[cid, pl.ds(r0, zr)],
                    gs[s]).wait()

    return seg_sum


def _full(shape):
    return pl.BlockSpec(shape, lambda i: tuple(0 for _ in shape))


def _row(blk, w):
    return pl.BlockSpec((blk, w), lambda i: (i, 0))


def _agg(blk):
    return pl.BlockSpec((2, blk, _W), lambda i: (0, i, 0))


@functools.lru_cache(maxsize=None)
def _make_tc1(n, blk):
    def body(x_ref, wl_ref, wr_ref, b_ref, y_ref, z_ref):
        xb = x_ref[...]
        y_ref[...] = jnp.dot(xb, wl_ref[...], preferred_element_type=_F32)
        z_ref[...] = jnp.dot(xb, wr_ref[...], preferred_element_type=_F32) + b_ref[...]

    return pl.pallas_call(
        body,
        grid=(n // blk,),
        in_specs=[_row(blk, 56), _full((56, 16)), _full((56, 16)), _full((1, 16))],
        out_specs=[_row(blk, 16), _row(blk, 16)],
        out_shape=[jax.ShapeDtypeStruct((n, 16), _F32),
                   jax.ShapeDtypeStruct((n, 16), _F32)],
    )


@functools.lru_cache(maxsize=None)
def _make_tc2(n, blk):
    def body(a_ref, c_ref, z_ref, h_ref, inv_ref):
        a = a_ref[0] + a_ref[1]
        cnt = c_ref[0, :, 0:1] + c_ref[1, :, 0:1]
        inv = 1.0 / jnp.maximum(cnt, 1.0)
        h_ref[...] = jnp.maximum(a * inv + z_ref[...], 0.0)
        inv_ref[...] = inv

    return pl.pallas_call(
        body,
        grid=(n // blk,),
        in_specs=[_agg(blk), _agg(blk), _row(blk, 16)],
        out_specs=[_row(blk, 16), _row(blk, 1)],
        out_shape=[jax.ShapeDtypeStruct((n, 16), _F32),
                   jax.ShapeDtypeStruct((n, 1), _F32)],
    )


@functools.lru_cache(maxsize=None)
def _make_tc3(n, blk):
    def body(a_ref, inv_ref, h_ref, wl_ref, wr_ref, b_ref, oa_ref, ob_ref):
        mean = (a_ref[0] + a_ref[1]) * inv_ref[...]
        o = jnp.maximum(
            jnp.dot(mean, wl_ref[...], preferred_element_type=_F32)
            + jnp.dot(h_ref[...], wr_ref[...], preferred_element_type=_F32)
            + b_ref[...], 0.0)
        oa_ref[...] = o[:, :16]
        ob_ref[...] = o[:, 16:]

    return pl.pallas_call(
        body,
        grid=(n // blk,),
        in_specs=[_agg(blk), _row(blk, 1), _row(blk, 16),
                  _full((16, 32)), _full((16, 32)), _full((1, 32))],
        out_specs=[_row(blk, 16), _row(blk, 16)],
        out_shape=[jax.ShapeDtypeStruct((n, 16), _F32),
                   jax.ShapeDtypeStruct((n, 16), _F32)],
    )


@functools.lru_cache(maxsize=None)
def _make_tc4(n, blk):
    def body(aa_ref, ab_ref, inv_ref, ha_ref, hb_ref, wl3_ref, wr3_ref,
             b3_ref, wl4_ref, wr4_ref, b4_ref, y_ref, z_ref):
        inv = inv_ref[...]
        mean = jnp.concatenate(
            [(aa_ref[0] + aa_ref[1]) * inv, (ab_ref[0] + ab_ref[1]) * inv],
            axis=1)
        h2 = jnp.concatenate([ha_ref[...], hb_ref[...]], axis=1)
        h3 = jnp.maximum(
            jnp.dot(mean, wl3_ref[...], preferred_element_type=_F32)
            + jnp.dot(h2, wr3_ref[...], preferred_element_type=_F32)
            + b3_ref[...], 0.0)
        pad = jnp.zeros((blk, 14), _F32)
        y_ref[...] = jnp.concatenate(
            [jnp.dot(h3, wl4_ref[...], preferred_element_type=_F32), pad], axis=1)
        z_ref[...] = jnp.concatenate(
            [jnp.dot(h3, wr4_ref[...], preferred_element_type=_F32) + b4_ref[...],
             pad], axis=1)

    return pl.pallas_call(
        body,
        grid=(n // blk,),
        in_specs=[_agg(blk), _agg(blk), _row(blk, 1), _row(blk, 16),
                  _row(blk, 16), _full((32, 64)), _full((32, 64)),
                  _full((1, 64)), _full((64, 2)), _full((64, 2)), _full((1, 2))],
        out_specs=[_row(blk, 16), _row(blk, 16)],
        out_shape=[jax.ShapeDtypeStruct((n, 16), _F32),
                   jax.ShapeDtypeStruct((n, 16), _F32)],
    )


@functools.lru_cache(maxsize=None)
def _make_tc5(n, blk):
    nb = n // blk

    def body(a_ref, inv_ref, z4_ref, bt_ref, o_ref, acc_ref):
        i = pl.program_id(0)
        a = a_ref[0] + a_ref[1]
        h4 = jnp.maximum(a[:, :2] * inv_ref[...] + z4_ref[:, :2], 0.0)
        hc = jnp.concatenate(
            [h4, jnp.ones((blk, 1), _F32), jnp.zeros((blk, 5), _F32)], axis=1)
        bt = bt_ref[...][0, 0]
        oh = (bt[:, None] == lax.broadcasted_iota(jnp.int32, (blk, _G), 1)
              ).astype(_F32)
        part = lax.dot_general(oh, hc, (((0,), (0,)), ((), ())),
                               preferred_element_type=_F32)

        @pl.when(i == 0)
        def _():
            acc_ref[...] = part

        @pl.when(i > 0)
        def _():
            acc_ref[...] = acc_ref[...] + part

        @pl.when(i == nb - 1)
        def _():
            acc = acc_ref[...]
            pooled = acc[:, :2] / jnp.maximum(acc[:, 2:3], 1.0)
            m = jnp.max(pooled, axis=1, keepdims=True)
            e = jnp.exp(pooled - m)
            o_ref[...] = e / jnp.sum(e, axis=1, keepdims=True)

    return pl.pallas_call(
        body,
        grid=(nb,),
        in_specs=[_agg(blk), _row(blk, 1), _row(blk, 16),
                  pl.BlockSpec((1, 1, blk), lambda i: (i, 0, 0))],
        out_specs=pl.BlockSpec((_G, 2), lambda i: (0, 0)),
        out_shape=jax.ShapeDtypeStruct((_G, 2), _F32),
        scratch_shapes=[pltpu.VMEM((_G, 8), _F32)],
    )


def kernel(x, edge_index, batch, Wl1, Wr1, b1, Wl2, Wr2, b2,
           Wl3, Wr3, b3, Wl4, Wr4, b4):
    n = x.shape[0]
    e = edge_index.shape[1]
    blk = next(d for d in range(min(2048, n), 7, -1) if n % d == 0 and d % 8 == 0)
    nb = n // blk

    k = -(-e // (_NW * _CH))
    ep = _NW * k * _CH
    src = jnp.concatenate(
        [edge_index[0], jnp.zeros((ep - e,), jnp.int32)]).reshape(_NW, k, _CH)
    dst = jnp.concatenate(
        [edge_index[1], jnp.full((ep - e,), n, jnp.int32)]).reshape(_NW, k, _CH)

    seg = _make_seg_sum(n, k, True)
    cnt = _make_seg_sum(n, k, False)

    y1, z1 = _make_tc1(n, blk)(x, Wl1, Wr1, b1.reshape(1, -1))
    c = cnt(y1, src, dst)
    a1 = seg(y1, src, dst)
    h1, inv = _make_tc2(n, blk)(a1, c, z1)
    a2 = seg(h1, src, dst)
    h2a, h2b = _make_tc3(n, blk)(a2, inv, h1, Wl2, Wr2, b2.reshape(1, -1))
    a3a = seg(h2a, src, dst)
    a3b = seg(h2b, src, dst)
    y4, z4 = _make_tc4(n, blk)(a3a, a3b, inv, h2a, h2b, Wl3, Wr3,
                               b3.reshape(1, -1), Wl4, Wr4, b4.reshape(1, -1))
    a4 = seg(y4, src, dst)
    return _make_tc5(n, blk)(a4, inv, z4, batch.reshape(nb, 1, blk))
